# baseline (speedup 1.0000x reference)
import functools

import jax
import jax.numpy as jnp
from jax import lax
from jax.experimental import pallas as pl
from jax.experimental.pallas import tpu as pltpu

_B = 16
_N0 = 1024
_NSAMPLE = 32


def _fps_body(xr, yr, zr, o1x, o1y, o1z, o2x, o2y, o2z, o3x, o3y, o3z,
              o4x, o4y, o4z, dmin_ref):
    iota128 = lax.broadcasted_iota(jnp.int32, (_B, 128), 1)

    def run_level(xref, yref, zref, n, npts, ox, oy, oz):
        nc = n // 128
        dmin_ref[:, :n] = jnp.full((_B, n), 1e10, jnp.float32)

        def step(s, carry):
            far, bx, by, bz = carry
            cc = lax.shift_right_logical(far, 7)
            lloc = jnp.bitwise_and(far, 127)
            cx = cy = cz = jnp.zeros((_B, 1), jnp.float32)
            for c in range(nc):
                sl = slice(c * 128, (c + 1) * 128)
                hit = cc == c
                cx = cx + jnp.where(
                    hit, jnp.take_along_axis(xref[:, sl], lloc, axis=1), 0.0)
                cy = cy + jnp.where(
                    hit, jnp.take_along_axis(yref[:, sl], lloc, axis=1), 0.0)
                cz = cz + jnp.where(
                    hit, jnp.take_along_axis(zref[:, sl], lloc, axis=1), 0.0)
            ins = iota128 == s
            bx = jnp.where(ins, cx, bx)
            by = jnp.where(ins, cy, by)
            bz = jnp.where(ins, cz, bz)
            best_v = jnp.full((_B, 1), -1.0, jnp.float32)
            best_i = jnp.full((_B, 1), n, jnp.int32)
            for c in range(nc):
                sl = slice(c * 128, (c + 1) * 128)
                d = ((xref[:, sl] - cx) ** 2 + (yref[:, sl] - cy) ** 2
                     + (zref[:, sl] - cz) ** 2)
                dm = jnp.minimum(dmin_ref[:, sl], d)
                dmin_ref[:, sl] = dm
                mv = jnp.max(dm, 1, keepdims=True)
                gi = iota128 + (c * 128)
                mi = jnp.min(jnp.where(dm == mv, gi, n), 1, keepdims=True)
                take = (mv > best_v) | ((mv == best_v) & (mi < best_i))
                best_v = jnp.where(take, mv, best_v)
                best_i = jnp.where(take, mi, best_i)
            return best_i, bx, by, bz

        far = jnp.zeros((_B, 1), jnp.int32)
        zbuf = jnp.zeros((_B, 128), jnp.float32)
        for chunk in range(npts // 128):
            far, bx, by, bz = lax.fori_loop(
                0, 128, step, (far, zbuf, zbuf, zbuf))
            ox[:, chunk * 128:(chunk + 1) * 128] = bx
            oy[:, chunk * 128:(chunk + 1) * 128] = by
            oz[:, chunk * 128:(chunk + 1) * 128] = bz

    run_level(xr, yr, zr, 1024, 1024, o1x, o1y, o1z)
    run_level(o1x, o1y, o1z, 1024, 512, o2x, o2y, o2z)
    run_level(o2x, o2y, o2z, 512, 256, o3x, o3y, o3z)
    run_level(o3x, o3y, o3z, 256, 128, o4x, o4y, o4z)


def _run_fps(l0):
    x0, y0, z0 = l0[:, :, 0], l0[:, :, 1], l0[:, :, 2]
    sizes = [1024, 1024, 1024, 512, 512, 512, 256, 256, 256, 128, 128, 128]
    outs = pl.pallas_call(
        _fps_body,
        out_shape=[jax.ShapeDtypeStruct((_B, s), jnp.float32) for s in sizes],
        scratch_shapes=[pltpu.VMEM((_B, 1024), jnp.float32)],
    )(x0, y0, z0)
    lvls = []
    for i in range(4):
        lvls.append(jnp.stack(outs[3 * i:3 * i + 3], axis=-1))
    return lvls


def _sa_body(xyz_ref, xyzT_ref, nxyz_ref, feats_ref,
             w1a_ref, w1b_ref, w2_ref, w3_ref, b1_ref, b2_ref, b3_ref,
             out_ref, rem_ref, *, r2, n, s, h3, has_feats):
    xyz = xyz_ref[0]
    xt = xyzT_ref[0]
    nx = nxyz_ref[0]
    ssq = jnp.sum(nx * nx, 1, keepdims=True)
    xsq = jnp.sum(xt * xt, 0, keepdims=True)
    sq = ssq + xsq - 2.0 * jnp.dot(nx, xt, preferred_element_type=jnp.float32)
    pred = ~(sq > r2)
    iota = lax.broadcasted_iota(jnp.int32, (s, n), 1)
    tot = jnp.sum(jnp.where(pred, 1.0, 0.0), 1, keepdims=True)
    trips = jnp.minimum(jnp.max(tot), 32.0).astype(jnp.int32)

    w1a = w1a_ref[...]
    w2 = w2_ref[...]
    w3 = w3_ref[...]
    b1 = b1_ref[...]
    b2 = b2_ref[...]
    b3 = b3_ref[...]
    if has_feats:
        feats = feats_ref[0]
        w1b = w1b_ref[...]

    rem_ref[...] = jnp.where(pred, iota, n)

    def body(k, acc):
        rem = rem_ref[...]
        mi = jnp.min(rem, 1, keepdims=True)
        oh = iota == mi
        sel = jnp.where(oh, 1.0, 0.0)
        rem_ref[...] = jnp.where(oh, n, rem)
        g3 = jnp.dot(sel, xyz, preferred_element_type=jnp.float32) - nx
        h = jnp.dot(g3, w1a, preferred_element_type=jnp.float32) + b1
        if has_feats:
            gc = jnp.dot(sel, feats, preferred_element_type=jnp.float32)
            h = h + jnp.dot(gc, w1b, preferred_element_type=jnp.float32)
        h = jnp.maximum(h, 0.0)
        h = jnp.maximum(jnp.dot(h, w2, preferred_element_type=jnp.float32) + b2, 0.0)
        h = jnp.maximum(jnp.dot(h, w3, preferred_element_type=jnp.float32) + b3, 0.0)
        h = jnp.where(tot >= (k + 1).astype(jnp.float32), h, 0.0)
        return jnp.maximum(acc, h)

    out_ref[0] = lax.fori_loop(0, trips, body,
                               jnp.zeros((s, h3), jnp.float32))


def _run_sa(xyz, new_xyz, feats, p, radius):
    b, n, _ = xyz.shape
    s = new_xyz.shape[1]
    ws, bs = p['w'], p['b']
    h1, h2, h3 = ws[0].shape[1], ws[1].shape[1], ws[2].shape[1]
    has_feats = feats is not None
    w1a = ws[0][:3]
    if has_feats:
        c = feats.shape[2]
        w1b = ws[0][3:]
    else:
        c = 1
        feats = jnp.zeros((b, n, 1), jnp.float32)
        w1b = jnp.zeros((1, h1), jnp.float32)
    xyzT = jnp.transpose(xyz, (0, 2, 1))
    body = functools.partial(_sa_body, r2=radius * radius, n=n, s=s, h3=h3,
                             has_feats=has_feats)
    full = lambda shp: pl.BlockSpec(shp, lambda i: (0,) * len(shp))
    out = pl.pallas_call(
        body,
        grid=(b,),
        in_specs=[
            pl.BlockSpec((1, n, 3), lambda i: (i, 0, 0)),
            pl.BlockSpec((1, 3, n), lambda i: (i, 0, 0)),
            pl.BlockSpec((1, s, 3), lambda i: (i, 0, 0)),
            pl.BlockSpec((1, n, c), lambda i: (i, 0, 0)),
            full(w1a.shape), full(w1b.shape), full(ws[1].shape),
            full(ws[2].shape), full((1, h1)), full((1, h2)), full((1, h3)),
        ],
        out_specs=pl.BlockSpec((1, s, h3), lambda i: (i, 0, 0)),
        out_shape=jax.ShapeDtypeStruct((b, s, h3), jnp.float32),
        scratch_shapes=[pltpu.VMEM((s, n), jnp.int32)],
    )(xyz, xyzT, new_xyz, feats, w1a, w1b, ws[1], ws[2],
      bs[0].reshape(1, h1), bs[1].reshape(1, h2), bs[2].reshape(1, h3))
    return out


def _fp_body(x1_ref, x2T_ref, f2_ref, w_ref, b_ref, out_ref, *, s2):
    x1 = x1_ref[0]
    x2t = x2T_ref[0]
    f2 = f2_ref[0]
    ssq = jnp.sum(x1 * x1, 1, keepdims=True)
    xsq = jnp.sum(x2t * x2t, 0, keepdims=True)
    d = ssq + xsq - 2.0 * jnp.dot(x1, x2t, preferred_element_type=jnp.float32)
    iota = lax.broadcasted_iota(jnp.int32, (_N0, s2), 1)
    wmat = jnp.zeros((_N0, s2), jnp.float32)
    recips = []
    onehots = []
    for _ in range(3):
        m = jnp.min(d, 1, keepdims=True)
        idx = jnp.min(jnp.where(d == m, iota, s2), 1, keepdims=True)
        oh = iota == idx
        recips.append(1.0 / (m + 1e-8))
        onehots.append(oh)
        d = jnp.where(oh, 1e30, d)
    norm = recips[0] + recips[1] + recips[2]
    for rc, oh in zip(recips, onehots):
        wmat = wmat + jnp.where(oh, rc / norm, 0.0)
    interp = jnp.dot(wmat, f2, preferred_element_type=jnp.float32)
    h = jnp.dot(interp, w_ref[...], preferred_element_type=jnp.float32) + b_ref[...]
    out_ref[0] = jnp.maximum(h, 0.0)


def _run_fp(xyz1, xyz2, feats2, p):
    b = xyz1.shape[0]
    s2, c2 = feats2.shape[1], feats2.shape[2]
    w = p['w'][0]
    ho = w.shape[1]
    x2T = jnp.transpose(xyz2, (0, 2, 1))
    body = functools.partial(_fp_body, s2=s2)
    full = lambda shp: pl.BlockSpec(shp, lambda i: (0,) * len(shp))
    return pl.pallas_call(
        body,
        grid=(b,),
        in_specs=[
            pl.BlockSpec((1, _N0, 3), lambda i: (i, 0, 0)),
            pl.BlockSpec((1, 3, s2), lambda i: (i, 0, 0)),
            pl.BlockSpec((1, s2, c2), lambda i: (i, 0, 0)),
            full(w.shape), full((1, ho)),
        ],
        out_specs=pl.BlockSpec((1, _N0, ho), lambda i: (i, 0, 0)),
        out_shape=jax.ShapeDtypeStruct((b, _N0, ho), jnp.float32),
    )(xyz1, x2T, feats2, w, p['b'][0].reshape(1, ho))


def _head_body(u4_ref, u3_ref, u2_ref, l1_ref, xyz_ref, lab_ref,
               wa_ref, wb_ref, wc_ref, wd_ref, we_ref, wlab_ref, b2_ref,
               w3a_ref, w3b_ref, b3_ref, w4a_ref, w4b_ref, b4_ref,
               w5_ref, b5_ref, out_ref):
    lab = lab_ref[0]
    dotf = lambda a, w: jnp.dot(a, w, preferred_element_type=jnp.float32)
    h = (dotf(u4_ref[0], wa_ref[...]) + dotf(u3_ref[0], wb_ref[...])
         + dotf(u2_ref[0], wc_ref[...]) + dotf(l1_ref[0], wd_ref[...])
         + dotf(xyz_ref[0], we_ref[...]) + dotf(lab, wlab_ref[...])
         + b2_ref[...])
    h = jnp.maximum(h, 0.0)
    h = jnp.maximum(dotf(h, w3a_ref[...]) + dotf(lab, w3b_ref[...])
                    + b3_ref[...], 0.0)
    h = jnp.maximum(dotf(h, w4a_ref[...]) + dotf(lab, w4b_ref[...])
                    + b4_ref[...], 0.0)
    h = jnp.maximum(dotf(h, w5_ref[...]) + b5_ref[...], 0.0)
    out_ref[0] = h


def _run_head(u4, u3, u2, l1p, xyz0, labels, params):
    b = xyz0.shape[0]
    w2 = params['c2']['w'][0]
    wa, wb, wc = w2[0:64], w2[104:168], w2[208:272]
    wd, we = w2[312:376], w2[376:379]
    wlab = w2[64:104] + w2[168:208] + w2[272:312]
    w3 = params['c3']['w'][0]
    w3a, w3b = w3[0:256], w3[256:296]
    w4 = params['c4']['w'][0]
    w4a, w4b = w4[0:128], w4[128:168]
    w5 = params['c5']['w'][0]
    lab3 = labels.reshape(b, 1, labels.shape[1])
    full = lambda shp: pl.BlockSpec(shp, lambda i: (0,) * len(shp))
    args = [u4, u3, u2, l1p, xyz0, lab3, wa, wb, wc, wd, we, wlab,
            params['c2']['b'][0].reshape(1, -1), w3a, w3b,
            params['c3']['b'][0].reshape(1, -1), w4a, w4b,
            params['c4']['b'][0].reshape(1, -1), w5,
            params['c5']['b'][0].reshape(1, -1)]
    in_specs = []
    for a in args[:6]:
        shp = (1,) + a.shape[1:]
        in_specs.append(pl.BlockSpec(shp, lambda i: (i, 0, 0)))
    for a in args[6:]:
        in_specs.append(full(a.shape))
    out = pl.pallas_call(
        _head_body,
        grid=(b,),
        in_specs=in_specs,
        out_specs=pl.BlockSpec((1, _N0, 3), lambda i: (i, 0, 0)),
        out_shape=jax.ShapeDtypeStruct((b, _N0, 3), jnp.float32),
    )(*args)
    return jnp.transpose(out, (0, 2, 1))


def kernel(point_cloud, labels_onehot, params):
    l0_xyz = point_cloud[:, :, :3]
    l1_xyz, l2_xyz, l3_xyz, l4_xyz = _run_fps(l0_xyz)
    l1_points = _run_sa(l0_xyz, l1_xyz, None, params['sa1'], 0.05)
    l2_points = _run_sa(l1_xyz, l2_xyz, l1_points, params['sa2'], 0.01)
    l3_points = _run_sa(l2_xyz, l3_xyz, l2_points, params['sa3'], 0.2)
    l4_points = _run_sa(l3_xyz, l4_xyz, l3_points, params['sa4'], 0.3)
    up_l4 = _run_fp(l0_xyz, l4_xyz, l4_points, params['fp1'])
    up_l3 = _run_fp(l0_xyz, l3_xyz, l3_points, params['fp2'])
    up_l2 = _run_fp(l0_xyz, l2_xyz, l2_points, params['fp3'])
    return _run_head(up_l4, up_l3, up_l2, l1_points, l0_xyz,
                     labels_onehot, params)


# speedup vs baseline: 1.1988x; 1.1988x over previous
import functools

import jax
import jax.numpy as jnp
from jax import lax
from jax.experimental import pallas as pl
from jax.experimental.pallas import tpu as pltpu

_B = 16
_N0 = 1024
_NSAMPLE = 32


def _fps_body(xr, yr, zr, o1x, o1y, o1z, o2x, o2y, o2z, o3x, o3y, o3z,
              o4x, o4y, o4z):
    def run_level(x, y, z, n, npts, ox, oy, oz):
        iota = lax.broadcasted_iota(jnp.int32, (_B, n), 1)
        iota_c = lax.broadcasted_iota(jnp.int32, (_B, 128), 1)

        def step(s, carry):
            dmin, far, bx, by, bz = carry
            mask = iota == far
            cx = jnp.sum(jnp.where(mask, x, 0.0), 1, keepdims=True)
            cy = jnp.sum(jnp.where(mask, y, 0.0), 1, keepdims=True)
            cz = jnp.sum(jnp.where(mask, z, 0.0), 1, keepdims=True)
            ins = iota_c == s
            bx = jnp.where(ins, cx, bx)
            by = jnp.where(ins, cy, by)
            bz = jnp.where(ins, cz, bz)
            d = (x - cx) ** 2 + (y - cy) ** 2 + (z - cz) ** 2
            dmin = jnp.minimum(dmin, d)
            m = jnp.max(dmin, 1, keepdims=True)
            far2 = jnp.min(jnp.where(dmin == m, iota, n), 1, keepdims=True)
            return dmin, far2, bx, by, bz

        dmin = jnp.full((_B, n), 1e10, jnp.float32)
        far = jnp.zeros((_B, 1), jnp.int32)
        zbuf = jnp.zeros((_B, 128), jnp.float32)
        for chunk in range(npts // 128):
            dmin, far, bx, by, bz = lax.fori_loop(
                0, 128, step, (dmin, far, zbuf, zbuf, zbuf))
            ox[:, chunk * 128:(chunk + 1) * 128] = bx
            oy[:, chunk * 128:(chunk + 1) * 128] = by
            oz[:, chunk * 128:(chunk + 1) * 128] = bz

    run_level(xr[...], yr[...], zr[...], 1024, 1024, o1x, o1y, o1z)
    run_level(o1x[...], o1y[...], o1z[...], 1024, 512, o2x, o2y, o2z)
    run_level(o2x[...], o2y[...], o2z[...], 512, 256, o3x, o3y, o3z)
    run_level(o3x[...], o3y[...], o3z[...], 256, 128, o4x, o4y, o4z)


def _run_fps(l0):
    x0, y0, z0 = l0[:, :, 0], l0[:, :, 1], l0[:, :, 2]
    sizes = [1024, 1024, 1024, 512, 512, 512, 256, 256, 256, 128, 128, 128]
    outs = pl.pallas_call(
        _fps_body,
        out_shape=[jax.ShapeDtypeStruct((_B, s), jnp.float32) for s in sizes],
    )(x0, y0, z0)
    lvls = []
    for i in range(4):
        lvls.append(jnp.stack(outs[3 * i:3 * i + 3], axis=-1))
    return lvls


def _sa_body(xyz_ref, xyzT_ref, nxyz_ref, feats_ref,
             w1a_ref, w1b_ref, w2_ref, w3_ref, b1_ref, b2_ref, b3_ref,
             out_ref, rem_ref, *, r2, n, s, h3, has_feats):
    xyz = xyz_ref[0]
    xt = xyzT_ref[0]
    nx = nxyz_ref[0]
    ssq = jnp.sum(nx * nx, 1, keepdims=True)
    xsq = jnp.sum(xt * xt, 0, keepdims=True)
    sq = ssq + xsq - 2.0 * jnp.dot(nx, xt, preferred_element_type=jnp.float32)
    pred = ~(sq > r2)
    iota = lax.broadcasted_iota(jnp.int32, (s, n), 1)
    tot = jnp.sum(jnp.where(pred, 1.0, 0.0), 1, keepdims=True)
    trips = jnp.minimum(jnp.max(tot), 32.0).astype(jnp.int32)

    w1a = w1a_ref[...]
    w2 = w2_ref[...]
    w3 = w3_ref[...]
    b1 = b1_ref[...]
    b2 = b2_ref[...]
    b3 = b3_ref[...]
    if has_feats:
        t1 = (jnp.dot(xyz, w1a, preferred_element_type=jnp.float32)
              + jnp.dot(feats_ref[0], w1b_ref[...],
                        preferred_element_type=jnp.float32))
        t1hi = t1.astype(jnp.bfloat16)
        t1lo = (t1 - t1hi.astype(jnp.float32)).astype(jnp.bfloat16)
        nxw = jnp.dot(nx, w1a, preferred_element_type=jnp.float32)

    rem_ref[...] = jnp.where(pred, iota, n)
    one_bf = jnp.ones((), jnp.bfloat16)
    zero_bf = jnp.zeros((), jnp.bfloat16)

    def body(k, acc):
        rem = rem_ref[...]
        mi = jnp.min(rem, 1, keepdims=True)
        oh = iota == mi
        rem_ref[...] = jnp.where(oh, n, rem)
        if has_feats:
            selb = jnp.where(oh, 1.0, 0.0).astype(jnp.bfloat16)
            h = (jnp.dot(selb, t1hi, preferred_element_type=jnp.float32)
                 + jnp.dot(selb, t1lo, preferred_element_type=jnp.float32)
                 - nxw + b1)
        else:
            sel = jnp.where(oh, 1.0, 0.0)
            g3 = jnp.dot(sel, xyz, preferred_element_type=jnp.float32) - nx
            h = jnp.dot(g3, w1a, preferred_element_type=jnp.float32) + b1
        h = jnp.maximum(h, 0.0)
        h = jnp.maximum(jnp.dot(h, w2, preferred_element_type=jnp.float32) + b2, 0.0)
        h = jnp.maximum(jnp.dot(h, w3, preferred_element_type=jnp.float32) + b3, 0.0)
        h = jnp.where(tot >= (k + 1).astype(jnp.float32), h, 0.0)
        return jnp.maximum(acc, h)

    out_ref[0] = lax.fori_loop(0, trips, body,
                               jnp.zeros((s, h3), jnp.float32))


def _run_sa(xyz, new_xyz, feats, p, radius):
    b, n, _ = xyz.shape
    s = new_xyz.shape[1]
    ws, bs = p['w'], p['b']
    h1, h2, h3 = ws[0].shape[1], ws[1].shape[1], ws[2].shape[1]
    has_feats = feats is not None
    w1a = ws[0][:3]
    if has_feats:
        c = feats.shape[2]
        w1b = ws[0][3:]
    else:
        c = 1
        feats = jnp.zeros((b, n, 1), jnp.float32)
        w1b = jnp.zeros((1, h1), jnp.float32)
    xyzT = jnp.transpose(xyz, (0, 2, 1))
    body = functools.partial(_sa_body, r2=radius * radius, n=n, s=s, h3=h3,
                             has_feats=has_feats)
    full = lambda shp: pl.BlockSpec(shp, lambda i: (0,) * len(shp))
    out = pl.pallas_call(
        body,
        grid=(b,),
        in_specs=[
            pl.BlockSpec((1, n, 3), lambda i: (i, 0, 0)),
            pl.BlockSpec((1, 3, n), lambda i: (i, 0, 0)),
            pl.BlockSpec((1, s, 3), lambda i: (i, 0, 0)),
            pl.BlockSpec((1, n, c), lambda i: (i, 0, 0)),
            full(w1a.shape), full(w1b.shape), full(ws[1].shape),
            full(ws[2].shape), full((1, h1)), full((1, h2)), full((1, h3)),
        ],
        out_specs=pl.BlockSpec((1, s, h3), lambda i: (i, 0, 0)),
        out_shape=jax.ShapeDtypeStruct((b, s, h3), jnp.float32),
        scratch_shapes=[pltpu.VMEM((s, n), jnp.int32)],
    )(xyz, xyzT, new_xyz, feats, w1a, w1b, ws[1], ws[2],
      bs[0].reshape(1, h1), bs[1].reshape(1, h2), bs[2].reshape(1, h3))
    return out


def _fp_body(x1_ref, x2T_ref, f2_ref, w_ref, b_ref, out_ref, *, s2):
    x1 = x1_ref[0]
    x2t = x2T_ref[0]
    f2 = f2_ref[0]
    ssq = jnp.sum(x1 * x1, 1, keepdims=True)
    xsq = jnp.sum(x2t * x2t, 0, keepdims=True)
    d = ssq + xsq - 2.0 * jnp.dot(x1, x2t, preferred_element_type=jnp.float32)
    iota = lax.broadcasted_iota(jnp.int32, (_N0, s2), 1)
    wmat = jnp.zeros((_N0, s2), jnp.float32)
    recips = []
    onehots = []
    for _ in range(3):
        m = jnp.min(d, 1, keepdims=True)
        idx = jnp.min(jnp.where(d == m, iota, s2), 1, keepdims=True)
        oh = iota == idx
        recips.append(1.0 / (m + 1e-8))
        onehots.append(oh)
        d = jnp.where(oh, 1e30, d)
    norm = recips[0] + recips[1] + recips[2]
    for rc, oh in zip(recips, onehots):
        wmat = wmat + jnp.where(oh, rc / norm, 0.0)
    interp = jnp.dot(wmat, f2, preferred_element_type=jnp.float32)
    h = jnp.dot(interp, w_ref[...], preferred_element_type=jnp.float32) + b_ref[...]
    out_ref[0] = jnp.maximum(h, 0.0)


def _run_fp(xyz1, xyz2, feats2, p):
    b = xyz1.shape[0]
    s2, c2 = feats2.shape[1], feats2.shape[2]
    w = p['w'][0]
    ho = w.shape[1]
    x2T = jnp.transpose(xyz2, (0, 2, 1))
    body = functools.partial(_fp_body, s2=s2)
    full = lambda shp: pl.BlockSpec(shp, lambda i: (0,) * len(shp))
    return pl.pallas_call(
        body,
        grid=(b,),
        in_specs=[
            pl.BlockSpec((1, _N0, 3), lambda i: (i, 0, 0)),
            pl.BlockSpec((1, 3, s2), lambda i: (i, 0, 0)),
            pl.BlockSpec((1, s2, c2), lambda i: (i, 0, 0)),
            full(w.shape), full((1, ho)),
        ],
        out_specs=pl.BlockSpec((1, _N0, ho), lambda i: (i, 0, 0)),
        out_shape=jax.ShapeDtypeStruct((b, _N0, ho), jnp.float32),
    )(xyz1, x2T, feats2, w, p['b'][0].reshape(1, ho))


def _head_body(u4_ref, u3_ref, u2_ref, l1_ref, xyz_ref, lab_ref,
               wa_ref, wb_ref, wc_ref, wd_ref, we_ref, wlab_ref, b2_ref,
               w3a_ref, w3b_ref, b3_ref, w4a_ref, w4b_ref, b4_ref,
               w5_ref, b5_ref, out_ref):
    lab = lab_ref[0]
    dotf = lambda a, w: jnp.dot(a, w, preferred_element_type=jnp.float32)
    h = (dotf(u4_ref[0], wa_ref[...]) + dotf(u3_ref[0], wb_ref[...])
         + dotf(u2_ref[0], wc_ref[...]) + dotf(l1_ref[0], wd_ref[...])
         + dotf(xyz_ref[0], we_ref[...]) + dotf(lab, wlab_ref[...])
         + b2_ref[...])
    h = jnp.maximum(h, 0.0)
    h = jnp.maximum(dotf(h, w3a_ref[...]) + dotf(lab, w3b_ref[...])
                    + b3_ref[...], 0.0)
    h = jnp.maximum(dotf(h, w4a_ref[...]) + dotf(lab, w4b_ref[...])
                    + b4_ref[...], 0.0)
    h = jnp.maximum(dotf(h, w5_ref[...]) + b5_ref[...], 0.0)
    out_ref[0] = h


def _run_head(u4, u3, u2, l1p, xyz0, labels, params):
    b = xyz0.shape[0]
    w2 = params['c2']['w'][0]
    wa, wb, wc = w2[0:64], w2[104:168], w2[208:272]
    wd, we = w2[312:376], w2[376:379]
    wlab = w2[64:104] + w2[168:208] + w2[272:312]
    w3 = params['c3']['w'][0]
    w3a, w3b = w3[0:256], w3[256:296]
    w4 = params['c4']['w'][0]
    w4a, w4b = w4[0:128], w4[128:168]
    w5 = params['c5']['w'][0]
    lab3 = labels.reshape(b, 1, labels.shape[1])
    full = lambda shp: pl.BlockSpec(shp, lambda i: (0,) * len(shp))
    args = [u4, u3, u2, l1p, xyz0, lab3, wa, wb, wc, wd, we, wlab,
            params['c2']['b'][0].reshape(1, -1), w3a, w3b,
            params['c3']['b'][0].reshape(1, -1), w4a, w4b,
            params['c4']['b'][0].reshape(1, -1), w5,
            params['c5']['b'][0].reshape(1, -1)]
    in_specs = []
    for a in args[:6]:
        shp = (1,) + a.shape[1:]
        in_specs.append(pl.BlockSpec(shp, lambda i: (i, 0, 0)))
    for a in args[6:]:
        in_specs.append(full(a.shape))
    out = pl.pallas_call(
        _head_body,
        grid=(b,),
        in_specs=in_specs,
        out_specs=pl.BlockSpec((1, _N0, 3), lambda i: (i, 0, 0)),
        out_shape=jax.ShapeDtypeStruct((b, _N0, 3), jnp.float32),
    )(*args)
    return jnp.transpose(out, (0, 2, 1))


def kernel(point_cloud, labels_onehot, params):
    l0_xyz = point_cloud[:, :, :3]
    l1_xyz, l2_xyz, l3_xyz, l4_xyz = _run_fps(l0_xyz)
    l1_points = _run_sa(l0_xyz, l1_xyz, None, params['sa1'], 0.05)
    l2_points = _run_sa(l1_xyz, l2_xyz, l1_points, params['sa2'], 0.01)
    l3_points = _run_sa(l2_xyz, l3_xyz, l2_points, params['sa3'], 0.2)
    l4_points = _run_sa(l3_xyz, l4_xyz, l3_points, params['sa4'], 0.3)
    up_l4 = _run_fp(l0_xyz, l4_xyz, l4_points, params['fp1'])
    up_l3 = _run_fp(l0_xyz, l3_xyz, l3_points, params['fp2'])
    up_l2 = _run_fp(l0_xyz, l2_xyz, l2_points, params['fp3'])
    return _run_head(up_l4, up_l3, up_l2, l1_points, l0_xyz,
                     labels_onehot, params)


# speedup vs baseline: 1.1992x; 1.0003x over previous
import functools

import jax
import jax.numpy as jnp
from jax import lax
from jax.experimental import pallas as pl
from jax.experimental.pallas import tpu as pltpu

_B = 16
_N0 = 1024
_NSAMPLE = 32


def _fps_body(xr, yr, zr, o1x, o1y, o1z, o2x, o2y, o2z, o3x, o3y, o3z,
              o4x, o4y, o4z):
    def run_level(x, y, z, n, npts, ox, oy, oz):
        iota = lax.broadcasted_iota(jnp.int32, (_B, n), 1)
        iota_c = lax.broadcasted_iota(jnp.int32, (_B, 128), 1)

        def step(s, carry):
            dmin, far, bx, by, bz = carry
            mask = iota == far
            cx = jnp.sum(jnp.where(mask, x, 0.0), 1, keepdims=True)
            cy = jnp.sum(jnp.where(mask, y, 0.0), 1, keepdims=True)
            cz = jnp.sum(jnp.where(mask, z, 0.0), 1, keepdims=True)
            ins = iota_c == s
            bx = jnp.where(ins, cx, bx)
            by = jnp.where(ins, cy, by)
            bz = jnp.where(ins, cz, bz)
            d = (x - cx) ** 2 + (y - cy) ** 2 + (z - cz) ** 2
            dmin = jnp.minimum(dmin, d)
            m = jnp.max(dmin, 1, keepdims=True)
            far2 = jnp.min(jnp.where(dmin == m, iota, n), 1, keepdims=True)
            return dmin, far2, bx, by, bz

        dmin = jnp.full((_B, n), 1e10, jnp.float32)
        far = jnp.zeros((_B, 1), jnp.int32)
        zbuf = jnp.zeros((_B, 128), jnp.float32)
        for chunk in range(npts // 128):
            dmin, far, bx, by, bz = lax.fori_loop(
                0, 128, step, (dmin, far, zbuf, zbuf, zbuf))
            ox[:, chunk * 128:(chunk + 1) * 128] = bx
            oy[:, chunk * 128:(chunk + 1) * 128] = by
            oz[:, chunk * 128:(chunk + 1) * 128] = bz

    run_level(xr[...], yr[...], zr[...], 1024, 1024, o1x, o1y, o1z)
    run_level(o1x[...], o1y[...], o1z[...], 1024, 512, o2x, o2y, o2z)
    run_level(o2x[...], o2y[...], o2z[...], 512, 256, o3x, o3y, o3z)
    run_level(o3x[...], o3y[...], o3z[...], 256, 128, o4x, o4y, o4z)


def _run_fps(l0):
    x0, y0, z0 = l0[:, :, 0], l0[:, :, 1], l0[:, :, 2]
    sizes = [1024, 1024, 1024, 512, 512, 512, 256, 256, 256, 128, 128, 128]
    outs = pl.pallas_call(
        _fps_body,
        out_shape=[jax.ShapeDtypeStruct((_B, s), jnp.float32) for s in sizes],
    )(x0, y0, z0)
    lvls = []
    for i in range(4):
        lvls.append(jnp.stack(outs[3 * i:3 * i + 3], axis=-1))
    return lvls


def _dot3x(a, b):
    return jnp.dot(a, b, preferred_element_type=jnp.float32)


def _sa_body(xyz_ref, xyzT_ref, nxyz_ref, feats_ref,
             w1a_ref, w1b_ref, w2_ref, w3_ref, b1_ref, b2_ref, b3_ref,
             out_ref, rem_ref, *, r2, n, s, h3, has_feats):
    xyz = xyz_ref[0]
    xt = xyzT_ref[0]
    nx = nxyz_ref[0]
    ssq = jnp.sum(nx * nx, 1, keepdims=True)
    xsq = jnp.sum(xt * xt, 0, keepdims=True)
    sq = ssq + xsq - 2.0 * _dot3x(nx, xt)
    pred = ~(sq > r2)
    iota = lax.broadcasted_iota(jnp.int32, (s, n), 1)
    tot = jnp.sum(jnp.where(pred, 1.0, 0.0), 1, keepdims=True)
    trips = jnp.minimum(jnp.max(tot), 32.0).astype(jnp.int32)

    w1a = w1a_ref[...]
    w2 = w2_ref[...]
    w3 = w3_ref[...]
    b1 = b1_ref[...]
    b2 = b2_ref[...]
    b3 = b3_ref[...]
    if has_feats:
        t1 = (jnp.dot(xyz, w1a, preferred_element_type=jnp.float32)
              + jnp.dot(feats_ref[0], w1b_ref[...],
                        preferred_element_type=jnp.float32))
        t1hi = t1.astype(jnp.bfloat16)
        t1lo = (t1 - t1hi.astype(jnp.float32)).astype(jnp.bfloat16)
        nxw = jnp.dot(nx, w1a, preferred_element_type=jnp.float32)

    rem_ref[...] = jnp.where(pred, iota, n)
    one_bf = jnp.ones((), jnp.bfloat16)
    zero_bf = jnp.zeros((), jnp.bfloat16)

    def body(k, acc):
        rem = rem_ref[...]
        mi = jnp.min(rem, 1, keepdims=True)
        oh = iota == jnp.minimum(mi, n - 1)
        rem_ref[...] = jnp.where(oh, n, rem)
        if has_feats:
            selb = jnp.where(oh, 1.0, 0.0).astype(jnp.bfloat16)
            h = (jnp.dot(selb, t1hi, preferred_element_type=jnp.float32)
                 + jnp.dot(selb, t1lo, preferred_element_type=jnp.float32)
                 - nxw + b1)
        else:
            sel = jnp.where(oh, 1.0, 0.0)
            g3 = jnp.dot(sel, xyz, preferred_element_type=jnp.float32) - nx
            h = jnp.dot(g3, w1a, preferred_element_type=jnp.float32) + b1
        h = jnp.maximum(h, 0.0)
        h = jnp.maximum(jnp.dot(h, w2, preferred_element_type=jnp.float32) + b2, 0.0)
        h = jnp.maximum(jnp.dot(h, w3, preferred_element_type=jnp.float32) + b3, 0.0)
        keep = (tot >= (k + 1).astype(jnp.float32)) | ((tot == 0.0) & (k == 0))
        h = jnp.where(keep, h, 0.0)
        return jnp.maximum(acc, h)

    out_ref[0] = lax.fori_loop(0, jnp.maximum(trips, 1), body,
                               jnp.zeros((s, h3), jnp.float32))


def _run_sa(xyz, new_xyz, feats, p, radius):
    b, n, _ = xyz.shape
    s = new_xyz.shape[1]
    ws, bs = p['w'], p['b']
    h1, h2, h3 = ws[0].shape[1], ws[1].shape[1], ws[2].shape[1]
    has_feats = feats is not None
    w1a = ws[0][:3]
    if has_feats:
        c = feats.shape[2]
        w1b = ws[0][3:]
    else:
        c = 1
        feats = jnp.zeros((b, n, 1), jnp.float32)
        w1b = jnp.zeros((1, h1), jnp.float32)
    xyzT = jnp.transpose(xyz, (0, 2, 1))
    body = functools.partial(_sa_body, r2=radius * radius, n=n, s=s, h3=h3,
                             has_feats=has_feats)
    full = lambda shp: pl.BlockSpec(shp, lambda i: (0,) * len(shp))
    out = pl.pallas_call(
        body,
        grid=(b,),
        in_specs=[
            pl.BlockSpec((1, n, 3), lambda i: (i, 0, 0)),
            pl.BlockSpec((1, 3, n), lambda i: (i, 0, 0)),
            pl.BlockSpec((1, s, 3), lambda i: (i, 0, 0)),
            pl.BlockSpec((1, n, c), lambda i: (i, 0, 0)),
            full(w1a.shape), full(w1b.shape), full(ws[1].shape),
            full(ws[2].shape), full((1, h1)), full((1, h2)), full((1, h3)),
        ],
        out_specs=pl.BlockSpec((1, s, h3), lambda i: (i, 0, 0)),
        out_shape=jax.ShapeDtypeStruct((b, s, h3), jnp.float32),
        scratch_shapes=[pltpu.VMEM((s, n), jnp.int32)],
    )(xyz, xyzT, new_xyz, feats, w1a, w1b, ws[1], ws[2],
      bs[0].reshape(1, h1), bs[1].reshape(1, h2), bs[2].reshape(1, h3))
    return out


def _fp_body(x1_ref, x2T_ref, f2_ref, w_ref, b_ref, out_ref, *, s2):
    f2 = f2_ref[0]
    iota = lax.broadcasted_iota(jnp.int32, (_N0, s2), 1)
    wmat = jnp.zeros((_N0, s2), jnp.float32)
    x1 = x1_ref[0]
    x2t = x2T_ref[0]
    ssq = jnp.sum(x1 * x1, 1, keepdims=True)
    xsq = jnp.sum(x2t * x2t, 0, keepdims=True)
    d = ssq + xsq - 2.0 * _dot3x(x1, x2t)
    recips = []
    onehots = []
    for _ in range(3):
        m = jnp.min(d, 1, keepdims=True)
        idx = jnp.min(jnp.where(d == m, iota, s2), 1, keepdims=True)
        oh = iota == idx
        recips.append(1.0 / (m + 1e-8))
        onehots.append(oh)
        d = jnp.where(oh, 1e30, d)
    norm = recips[0] + recips[1] + recips[2]
    for rc, oh in zip(recips, onehots):
        wmat = wmat + jnp.where(oh, rc / norm, 0.0)
    interp = jnp.dot(wmat, f2, preferred_element_type=jnp.float32)
    h = jnp.dot(interp, w_ref[...], preferred_element_type=jnp.float32) + b_ref[...]
    out_ref[0] = jnp.maximum(h, 0.0)


def _run_fp(xyz1, xyz2, feats2, p):
    b = xyz1.shape[0]
    s2, c2 = feats2.shape[1], feats2.shape[2]
    w = p['w'][0]
    ho = w.shape[1]
    x2T = jnp.transpose(xyz2, (0, 2, 1))
    body = functools.partial(_fp_body, s2=s2)
    full = lambda shp: pl.BlockSpec(shp, lambda i: (0,) * len(shp))
    return pl.pallas_call(
        body,
        grid=(b,),
        in_specs=[
            pl.BlockSpec((1, _N0, 3), lambda i: (i, 0, 0)),
            pl.BlockSpec((1, 3, s2), lambda i: (i, 0, 0)),
            pl.BlockSpec((1, s2, c2), lambda i: (i, 0, 0)),
            full(w.shape), full((1, ho)),
        ],
        out_specs=pl.BlockSpec((1, _N0, ho), lambda i: (i, 0, 0)),
        out_shape=jax.ShapeDtypeStruct((b, _N0, ho), jnp.float32),
    )(xyz1, x2T, feats2, w, p['b'][0].reshape(1, ho))


def _head_body(u4_ref, u3_ref, u2_ref, l1_ref, xyz_ref, lab_ref,
               wa_ref, wb_ref, wc_ref, wd_ref, we_ref, wlab_ref, b2_ref,
               w3a_ref, w3b_ref, b3_ref, w4a_ref, w4b_ref, b4_ref,
               w5_ref, b5_ref, out_ref):
    lab = lab_ref[0]
    dotf = lambda a, w: jnp.dot(a, w, preferred_element_type=jnp.float32)
    h = (dotf(u4_ref[0], wa_ref[...]) + dotf(u3_ref[0], wb_ref[...])
         + dotf(u2_ref[0], wc_ref[...]) + dotf(l1_ref[0], wd_ref[...])
         + dotf(xyz_ref[0], we_ref[...]) + dotf(lab, wlab_ref[...])
         + b2_ref[...])
    h = jnp.maximum(h, 0.0)
    h = jnp.maximum(dotf(h, w3a_ref[...]) + dotf(lab, w3b_ref[...])
                    + b3_ref[...], 0.0)
    h = jnp.maximum(dotf(h, w4a_ref[...]) + dotf(lab, w4b_ref[...])
                    + b4_ref[...], 0.0)
    h = jnp.maximum(dotf(h, w5_ref[...]) + b5_ref[...], 0.0)
    out_ref[0] = h


def _run_head(u4, u3, u2, l1p, xyz0, labels, params):
    b = xyz0.shape[0]
    w2 = params['c2']['w'][0]
    wa, wb, wc = w2[0:64], w2[104:168], w2[208:272]
    wd, we = w2[312:376], w2[376:379]
    wlab = w2[64:104] + w2[168:208] + w2[272:312]
    w3 = params['c3']['w'][0]
    w3a, w3b = w3[0:256], w3[256:296]
    w4 = params['c4']['w'][0]
    w4a, w4b = w4[0:128], w4[128:168]
    w5 = params['c5']['w'][0]
    lab3 = labels.reshape(b, 1, labels.shape[1])
    full = lambda shp: pl.BlockSpec(shp, lambda i: (0,) * len(shp))
    args = [u4, u3, u2, l1p, xyz0, lab3, wa, wb, wc, wd, we, wlab,
            params['c2']['b'][0].reshape(1, -1), w3a, w3b,
            params['c3']['b'][0].reshape(1, -1), w4a, w4b,
            params['c4']['b'][0].reshape(1, -1), w5,
            params['c5']['b'][0].reshape(1, -1)]
    in_specs = []
    for a in args[:6]:
        shp = (1,) + a.shape[1:]
        in_specs.append(pl.BlockSpec(shp, lambda i: (i, 0, 0)))
    for a in args[6:]:
        in_specs.append(full(a.shape))
    out = pl.pallas_call(
        _head_body,
        grid=(b,),
        in_specs=in_specs,
        out_specs=pl.BlockSpec((1, _N0, 3), lambda i: (i, 0, 0)),
        out_shape=jax.ShapeDtypeStruct((b, _N0, 3), jnp.float32),
    )(*args)
    return jnp.transpose(out, (0, 2, 1))


def kernel(point_cloud, labels_onehot, params):
    l0_xyz = point_cloud[:, :, :3]
    l1_xyz, l2_xyz, l3_xyz, l4_xyz = _run_fps(l0_xyz)
    l1_points = _run_sa(l0_xyz, l1_xyz, None, params['sa1'], 0.05)
    l2_points = _run_sa(l1_xyz, l2_xyz, l1_points, params['sa2'], 0.01)
    l3_points = _run_sa(l2_xyz, l3_xyz, l2_points, params['sa3'], 0.2)
    l4_points = _run_sa(l3_xyz, l4_xyz, l3_points, params['sa4'], 0.3)
    up_l4 = _run_fp(l0_xyz, l4_xyz, l4_points, params['fp1'])
    up_l3 = _run_fp(l0_xyz, l3_xyz, l3_points, params['fp2'])
    up_l2 = _run_fp(l0_xyz, l2_xyz, l2_points, params['fp3'])
    return _run_head(up_l4, up_l3, up_l2, l1_points, l0_xyz,
                     labels_onehot, params)
